# baseline (speedup 1.0000x reference)
import functools

import jax
import jax.numpy as jnp
from jax import lax
from jax.experimental import pallas as pl
from jax.experimental.pallas import tpu as pltpu
from jax.experimental.pallas import tpu_sc as plsc

_EPS = 1e-15
_D = 128
_NC, _NS = 2, 16
_NW = _NC * _NS
_CHUNK = 80
_LANES = 16
_NACC = 4


def _dots_sc(z, src_idx, dst_idx):
    e_total = src_idx.shape[0]
    e_per_w = e_total // _NW
    n_chunks = e_per_w // _CHUNK
    mesh = plsc.VectorSubcoreMesh(core_axis_name="c", subcore_axis_name="s")

    @functools.partial(
        pl.kernel,
        out_type=jax.ShapeDtypeStruct((e_total,), jnp.float32),
        mesh=mesh,
        scratch_types=[
            pltpu.VMEM((e_per_w,), jnp.int32),
            pltpu.VMEM((e_per_w,), jnp.int32),
            pltpu.VMEM((e_per_w + _LANES,), jnp.float32),
            pltpu.VMEM((2, _CHUNK, _D), jnp.float32),
            pltpu.VMEM((2, _CHUNK, _D), jnp.float32),
            pltpu.SemaphoreType.DMA,
            pltpu.SemaphoreType.DMA,
            pltpu.SemaphoreType.DMA,
            pltpu.SemaphoreType.DMA,
        ],
        compiler_params=pltpu.CompilerParams(needs_layout_passes=False),
    )
    def dots_kernel(z_hbm, src_hbm, dst_hbm, out_hbm,
                    sidx_v, didx_v, out_v, srows, drows,
                    sem_s0, sem_d0, sem_s1, sem_d1):
        sems = ((sem_s0, sem_d0), (sem_s1, sem_d1))
        wid = lax.axis_index("s") * _NC + lax.axis_index("c")
        base = pl.multiple_of(wid * e_per_w, 8)
        pltpu.sync_copy(src_hbm.at[pl.ds(base, e_per_w)], sidx_v)
        pltpu.sync_copy(dst_hbm.at[pl.ds(base, e_per_w)], didx_v)

        def gathers(c, b):
            off = pl.multiple_of(c * _CHUNK, 8)
            return (
                pltpu.make_async_copy(
                    z_hbm.at[sidx_v.at[pl.ds(off, _CHUNK)]], srows.at[b],
                    sems[b][0]),
                pltpu.make_async_copy(
                    z_hbm.at[didx_v.at[pl.ds(off, _CHUNK)]], drows.at[b],
                    sems[b][1]),
            )

        def issue(c, b):
            for cp in gathers(c, b):
                cp.start()

        def wait(c, b):
            for cp in gathers(c, b):
                cp.wait()

        iota16 = lax.iota(jnp.int32, _LANES)
        lane0 = iota16 == 0

        def compute(c, b):
            off = c * _CHUNK
            sb = srows.at[b]
            db = drows.at[b]

            @plsc.parallel_loop(0, _CHUNK, step=1, unroll=4)
            def edge(e):
                a0 = sb[e, pl.ds(0, _LANES)] * db[e, pl.ds(0, _LANES)]
                a1 = (sb[e, pl.ds(_LANES, _LANES)]
                      * db[e, pl.ds(_LANES, _LANES)])
                for k in range(2, _D // _LANES, 2):
                    a0 = a0 + (sb[e, pl.ds(k * _LANES, _LANES)]
                               * db[e, pl.ds(k * _LANES, _LANES)])
                    a1 = a1 + (sb[e, pl.ds((k + 1) * _LANES, _LANES)]
                               * db[e, pl.ds((k + 1) * _LANES, _LANES)])
                p = a0 + a1
                for h in (8, 4, 2, 1):
                    p = p + jnp.take_along_axis(p, iota16 ^ h, axis=0)
                plsc.store_compressed(out_v.at[pl.ds(off + e, _LANES)],
                                      p, mask=lane0)

        issue(0, 0)

        def body(i, _):
            c0 = 2 * i
            issue(c0 + 1, 1)
            wait(c0, 0)
            compute(c0, 0)

            @pl.when(c0 + 2 < n_chunks)
            def _():
                issue(c0 + 2, 0)

            wait(c0 + 1, 1)
            compute(c0 + 1, 1)
            return 0

        lax.fori_loop(0, n_chunks // 2, body, 0)
        pltpu.sync_copy(out_v.at[pl.ds(0, e_per_w)],
                        out_hbm.at[pl.ds(base, e_per_w)])

    return dots_kernel(z, src_idx, dst_idx)


def _loss_tc(dots2d, n_per_side):
    rows = dots2d.shape[0]

    def body(d_ref, out_ref):
        v = d_ref[...]
        pos = v[: rows // 2]
        neg = v[rows // 2:]
        p = jax.nn.sigmoid(pos)
        q = jax.nn.sigmoid(neg)
        total = (jnp.sum(-jnp.log(p + _EPS))
                 + jnp.sum(-jnp.log(1.0 - q + _EPS)))
        out_ref[0, 0] = total / jnp.float32(n_per_side)

    return pl.pallas_call(
        body,
        out_shape=jax.ShapeDtypeStruct((1, 1), jnp.float32),
        in_specs=[pl.BlockSpec(memory_space=pltpu.VMEM)],
        out_specs=pl.BlockSpec(memory_space=pltpu.SMEM),
    )(dots2d)


def kernel(z, pos_edge_index, neg_edge_index):
    n = pos_edge_index.shape[1]
    src = jnp.concatenate([pos_edge_index[0], neg_edge_index[0]])
    dst = jnp.concatenate([pos_edge_index[1], neg_edge_index[1]])
    dots = _dots_sc(z, src, dst)
    loss = _loss_tc(dots.reshape(2 * n // _D, _D), n)
    return loss[0, 0]


# speedup vs baseline: 1.0424x; 1.0424x over previous
import functools

import jax
import jax.numpy as jnp
from jax import lax
from jax.experimental import pallas as pl
from jax.experimental.pallas import tpu as pltpu
from jax.experimental.pallas import tpu_sc as plsc

_EPS = 1e-15
_D = 128
_NC, _NS = 2, 16
_NW = _NC * _NS
_CHUNK = 80
_LANES = 16
_NACC = 4


def _dots_sc(z, src_idx, dst_idx):
    e_total = src_idx.shape[0]
    e_per_w = e_total // _NW
    n_chunks = e_per_w // _CHUNK
    mesh = plsc.VectorSubcoreMesh(core_axis_name="c", subcore_axis_name="s")

    @functools.partial(
        pl.kernel,
        out_type=jax.ShapeDtypeStruct((e_total,), jnp.float32),
        mesh=mesh,
        scratch_types=[
            pltpu.VMEM((e_per_w,), jnp.int32),
            pltpu.VMEM((e_per_w,), jnp.int32),
            pltpu.VMEM((e_per_w + _LANES,), jnp.float32),
            pltpu.VMEM((2, _CHUNK, _D), jnp.float32),
            pltpu.VMEM((2, _CHUNK, _D), jnp.float32),
            pltpu.SemaphoreType.DMA,
            pltpu.SemaphoreType.DMA,
            pltpu.SemaphoreType.DMA,
            pltpu.SemaphoreType.DMA,
        ],
        compiler_params=pltpu.CompilerParams(needs_layout_passes=False),
    )
    def dots_kernel(z_hbm, src_hbm, dst_hbm, out_hbm,
                    sidx_v, didx_v, out_v, srows, drows,
                    sem_s0, sem_d0, sem_s1, sem_d1):
        sems = ((sem_s0, sem_d0), (sem_s1, sem_d1))
        wid = lax.axis_index("s") * _NC + lax.axis_index("c")
        base = pl.multiple_of(wid * e_per_w, 8)
        pltpu.sync_copy(src_hbm.at[pl.ds(base, e_per_w)], sidx_v)
        pltpu.sync_copy(dst_hbm.at[pl.ds(base, e_per_w)], didx_v)

        def gathers(c, b):
            off = pl.multiple_of(c * _CHUNK, 8)
            return (
                pltpu.make_async_copy(
                    z_hbm.at[sidx_v.at[pl.ds(off, _CHUNK)]], srows.at[b],
                    sems[b][0]),
                pltpu.make_async_copy(
                    z_hbm.at[didx_v.at[pl.ds(off, _CHUNK)]], drows.at[b],
                    sems[b][1]),
            )

        def issue(c, b):
            for cp in gathers(c, b):
                cp.start()

        def wait(c, b):
            for cp in gathers(c, b):
                cp.wait()

        iota16 = lax.iota(jnp.int32, _LANES)
        lane0 = iota16 == 0

        def compute(c, b):
            off = c * _CHUNK
            sb = srows.at[b]
            db = drows.at[b]

            @plsc.parallel_loop(0, _CHUNK, step=1, unroll=2)
            def edge(e):
                a0 = sb[e, pl.ds(0, _LANES)] * db[e, pl.ds(0, _LANES)]
                a1 = (sb[e, pl.ds(_LANES, _LANES)]
                      * db[e, pl.ds(_LANES, _LANES)])
                for k in range(2, _D // _LANES, 2):
                    a0 = a0 + (sb[e, pl.ds(k * _LANES, _LANES)]
                               * db[e, pl.ds(k * _LANES, _LANES)])
                    a1 = a1 + (sb[e, pl.ds((k + 1) * _LANES, _LANES)]
                               * db[e, pl.ds((k + 1) * _LANES, _LANES)])
                p = a0 + a1
                for h in (8, 4, 2, 1):
                    p = p + jnp.take_along_axis(p, iota16 ^ h, axis=0)
                plsc.store_compressed(out_v.at[pl.ds(off + e, _LANES)],
                                      p, mask=lane0)

        issue(0, 0)

        def body(i, _):
            c0 = 2 * i
            issue(c0 + 1, 1)
            wait(c0, 0)
            compute(c0, 0)

            @pl.when(c0 + 2 < n_chunks)
            def _():
                issue(c0 + 2, 0)

            wait(c0 + 1, 1)
            compute(c0 + 1, 1)
            return 0

        lax.fori_loop(0, n_chunks // 2, body, 0)
        pltpu.sync_copy(out_v.at[pl.ds(0, e_per_w)],
                        out_hbm.at[pl.ds(base, e_per_w)])

    return dots_kernel(z, src_idx, dst_idx)


def _loss_tc(dots2d, n_per_side):
    rows = dots2d.shape[0]

    def body(d_ref, out_ref):
        v = d_ref[...]
        pos = v[: rows // 2]
        neg = v[rows // 2:]
        p = jax.nn.sigmoid(pos)
        q = jax.nn.sigmoid(neg)
        total = (jnp.sum(-jnp.log(p + _EPS))
                 + jnp.sum(-jnp.log(1.0 - q + _EPS)))
        out_ref[0, 0] = total / jnp.float32(n_per_side)

    return pl.pallas_call(
        body,
        out_shape=jax.ShapeDtypeStruct((1, 1), jnp.float32),
        in_specs=[pl.BlockSpec(memory_space=pltpu.VMEM)],
        out_specs=pl.BlockSpec(memory_space=pltpu.SMEM),
    )(dots2d)


def kernel(z, pos_edge_index, neg_edge_index):
    n = pos_edge_index.shape[1]
    src = jnp.concatenate([pos_edge_index[0], neg_edge_index[0]])
    dst = jnp.concatenate([pos_edge_index[1], neg_edge_index[1]])
    dots = _dots_sc(z, src, dst)
    loss = _loss_tc(dots.reshape(2 * n // _D, _D), n)
    return loss[0, 0]


# speedup vs baseline: 1.1082x; 1.0632x over previous
import functools

import jax
import jax.numpy as jnp
from jax import lax
from jax.experimental import pallas as pl
from jax.experimental.pallas import tpu as pltpu
from jax.experimental.pallas import tpu_sc as plsc

_EPS = 1e-15
_D = 128
_NC, _NS = 2, 16
_NW = _NC * _NS
_CHUNK = 80
_LANES = 16
_NACC = 4


def _dots_sc(z, pos_ei, neg_ei):
    e_total = 2 * pos_ei.shape[1]
    e_per_w = e_total // _NW
    n_chunks = e_per_w // _CHUNK
    mesh = plsc.VectorSubcoreMesh(core_axis_name="c", subcore_axis_name="s")

    @functools.partial(
        pl.kernel,
        out_type=jax.ShapeDtypeStruct((e_total,), jnp.float32),
        mesh=mesh,
        scratch_types=[
            pltpu.VMEM((e_per_w,), jnp.int32),
            pltpu.VMEM((e_per_w,), jnp.int32),
            pltpu.VMEM((e_per_w + _LANES,), jnp.float32),
            pltpu.VMEM((2, _CHUNK, _D), jnp.float32),
            pltpu.VMEM((2, _CHUNK, _D), jnp.float32),
            pltpu.SemaphoreType.DMA,
            pltpu.SemaphoreType.DMA,
            pltpu.SemaphoreType.DMA,
            pltpu.SemaphoreType.DMA,
        ],
        compiler_params=pltpu.CompilerParams(needs_layout_passes=False),
    )
    def dots_kernel(z_hbm, pos_hbm, neg_hbm, out_hbm,
                    sidx_v, didx_v, out_v, srows, drows,
                    sem_s0, sem_d0, sem_s1, sem_d1):
        sems = ((sem_s0, sem_d0), (sem_s1, sem_d1))
        wid = lax.axis_index("s") * _NC + lax.axis_index("c")
        base = pl.multiple_of(wid * e_per_w, 8)
        half_w = _NW // 2
        half_e = e_total // 2

        @pl.when(wid < half_w)
        def _():
            off = pl.multiple_of(wid * e_per_w, 8)
            pltpu.sync_copy(pos_hbm.at[pl.ds(off, e_per_w)], sidx_v)
            pltpu.sync_copy(pos_hbm.at[pl.ds(half_e + off, e_per_w)], didx_v)

        @pl.when(wid >= half_w)
        def _():
            off = pl.multiple_of((wid - half_w) * e_per_w, 8)
            pltpu.sync_copy(neg_hbm.at[pl.ds(off, e_per_w)], sidx_v)
            pltpu.sync_copy(neg_hbm.at[pl.ds(half_e + off, e_per_w)], didx_v)

        def gathers(c, b):
            off = pl.multiple_of(c * _CHUNK, 8)
            return (
                pltpu.make_async_copy(
                    z_hbm.at[sidx_v.at[pl.ds(off, _CHUNK)]], srows.at[b],
                    sems[b][0]),
                pltpu.make_async_copy(
                    z_hbm.at[didx_v.at[pl.ds(off, _CHUNK)]], drows.at[b],
                    sems[b][1]),
            )

        def issue(c, b):
            for cp in gathers(c, b):
                cp.start()

        def wait(c, b):
            for cp in gathers(c, b):
                cp.wait()

        iota16 = lax.iota(jnp.int32, _LANES)
        lane0 = iota16 == 0

        def compute(c, b):
            off = c * _CHUNK
            sb = srows.at[b]
            db = drows.at[b]

            @plsc.parallel_loop(0, _CHUNK, step=1, unroll=2)
            def edge(e):
                a0 = sb[e, pl.ds(0, _LANES)] * db[e, pl.ds(0, _LANES)]
                a1 = (sb[e, pl.ds(_LANES, _LANES)]
                      * db[e, pl.ds(_LANES, _LANES)])
                for k in range(2, _D // _LANES, 2):
                    a0 = a0 + (sb[e, pl.ds(k * _LANES, _LANES)]
                               * db[e, pl.ds(k * _LANES, _LANES)])
                    a1 = a1 + (sb[e, pl.ds((k + 1) * _LANES, _LANES)]
                               * db[e, pl.ds((k + 1) * _LANES, _LANES)])
                p = a0 + a1
                for h in (8, 4, 2, 1):
                    p = p + jnp.take_along_axis(p, iota16 ^ h, axis=0)
                plsc.store_compressed(out_v.at[pl.ds(off + e, _LANES)],
                                      p, mask=lane0)

        issue(0, 0)

        def body(i, _):
            c0 = 2 * i
            issue(c0 + 1, 1)
            wait(c0, 0)
            compute(c0, 0)

            @pl.when(c0 + 2 < n_chunks)
            def _():
                issue(c0 + 2, 0)

            wait(c0 + 1, 1)
            compute(c0 + 1, 1)
            return 0

        lax.fori_loop(0, n_chunks // 2, body, 0)
        pltpu.sync_copy(out_v.at[pl.ds(0, e_per_w)],
                        out_hbm.at[pl.ds(base, e_per_w)])

    return dots_kernel(z, pos_ei.reshape(-1), neg_ei.reshape(-1))


def _loss_tc(dots2d, n_per_side):
    rows = dots2d.shape[0]

    def body(d_ref, out_ref):
        v = d_ref[...]
        pos = v[: rows // 2]
        neg = v[rows // 2:]
        p = jax.nn.sigmoid(pos)
        q = jax.nn.sigmoid(neg)
        total = (jnp.sum(-jnp.log(p + _EPS))
                 + jnp.sum(-jnp.log(1.0 - q + _EPS)))
        out_ref[0, 0] = total / jnp.float32(n_per_side)

    return pl.pallas_call(
        body,
        out_shape=jax.ShapeDtypeStruct((1, 1), jnp.float32),
        in_specs=[pl.BlockSpec(memory_space=pltpu.VMEM)],
        out_specs=pl.BlockSpec(memory_space=pltpu.SMEM),
    )(dots2d)


def kernel(z, pos_edge_index, neg_edge_index):
    n = pos_edge_index.shape[1]
    dots = _dots_sc(z, pos_edge_index, neg_edge_index)
    loss = _loss_tc(dots.reshape(2 * n // _D, _D), n)
    return loss[0, 0]
